# baseline (speedup 1.0000x reference)
import jax
import jax.numpy as jnp
from jax.experimental import pallas as pl

N = 10000
E = 20000
B = 128
D = 64
N_MP = 6
N_S2S = 6
N_LSTM = 3


def _mpnn(p, x, ef, src, dst):
    h = jax.nn.relu(x @ p['Wp'] + p['bp'])
    we = (jax.nn.relu(ef @ p['We1'] + p['be1']) @ p['We2'] + p['be2']).reshape(E, D, D)
    hidden = h
    node = h
    for _ in range(N_MP):
        msg = jnp.einsum('ei,eio->eo', node[src], we)
        agg = jax.ops.segment_sum(msg, dst, num_segments=N) + p['bconv']
        node = jax.nn.relu(agg)
        gi = node @ p['Wih'].T + p['bih']
        gh = hidden @ p['Whh'].T + p['bhh']
        ir, iz, inn = jnp.split(gi, 3, axis=-1)
        hr, hz, hn = jnp.split(gh, 3, axis=-1)
        r = jax.nn.sigmoid(ir + hr)
        z = jax.nn.sigmoid(iz + hz)
        n = jnp.tanh(inn + r * hn)
        hidden = (1.0 - z) * n + z * hidden
        node = hidden
    return node


def _set2set(p, feat, gid):
    q_star = jnp.zeros((B, 2 * D), jnp.float32)
    h = [jnp.zeros((B, D), jnp.float32) for _ in range(N_LSTM)]
    c = [jnp.zeros((B, D), jnp.float32) for _ in range(N_LSTM)]
    for _ in range(N_S2S):
        inp = q_star
        nh, nc = [], []
        for l in range(N_LSTM):
            g = inp @ p['Wih_%d' % l].T + p['bih_%d' % l] + h[l] @ p['Whh_%d' % l].T + p['bhh_%d' % l]
            ii, ff, gg, oo = jnp.split(g, 4, axis=-1)
            cc = jax.nn.sigmoid(ff) * c[l] + jax.nn.sigmoid(ii) * jnp.tanh(gg)
            hh = jax.nn.sigmoid(oo) * jnp.tanh(cc)
            nh.append(hh)
            nc.append(cc)
            inp = hh
        h, c = nh, nc
        q = h[-1]
        e = jnp.sum(feat * q[gid], axis=-1)
        emax = jax.ops.segment_max(e, gid, num_segments=B)
        ex = jnp.exp(e - emax[gid])
        den = jax.ops.segment_sum(ex, gid, num_segments=B)
        alpha = ex / den[gid]
        readout = jax.ops.segment_sum(alpha[:, None] * feat, gid, num_segments=B)
        q_star = jnp.concatenate([q, readout], axis=-1)
    return q_star


def _copy_k(x_ref, o_ref):
    o_ref[...] = x_ref[...]


def kernel(x_solu, e_solu, x_a1, e_a1, x_a2, e_a2, x_b1, e_b1, x_b2, e_b2, facs_a, facs_b, params, ei_solu, gid_solu, ei_a1, gid_a1, ei_a2, gid_a2, ei_b1, gid_b1, ei_b2, gid_b2):
    def gfeat(gp, x, ef, ei, gid):
        node = _mpnn(gp, x, ef, ei[0], ei[1])
        return _set2set(params['s2s'], node, gid)
    g_solu = gfeat(params['solu'], x_solu, e_solu, ei_solu, gid_solu)
    ga1 = gfeat(params['solv_a'], x_a1, e_a1, ei_a1, gid_a1)
    ga2 = gfeat(params['solv_a'], x_a2, e_a2, ei_a2, gid_a2)
    ga = facs_a[:, 0:1] * ga1 + facs_a[:, 1:2] * ga2
    gb1 = gfeat(params['solv_b'], x_b1, e_b1, ei_b1, gid_b1)
    gb2 = gfeat(params['solv_b'], x_b2, e_b2, ei_b2, gid_b2)
    gb = facs_b[:, 0:1] * gb1 + facs_b[:, 1:2] * gb2
    hcat = jnp.concatenate([g_solu, ga, gb], axis=-1)
    pr = params['pred']
    hid = jax.nn.relu(hcat @ pr['W1'] + pr['b1'])
    out = hid @ pr['W2'] + pr['b2']
    return pl.pallas_call(
        _copy_k, out_shape=jax.ShapeDtypeStruct(out.shape, out.dtype))(out)


# speedup vs baseline: 3.0958x; 3.0958x over previous
import functools

import jax
import jax.numpy as jnp
from jax import lax
from jax.experimental import pallas as pl
from jax.experimental.pallas import tpu as pltpu
from jax.experimental.pallas import tpu_sc as plsc

N = 10000
E = 20000
B = 128
D = 64
D_NODE = 128
D_EH = 128
N_MP = 6
N_S2S = 6

EP = 20480
R = 5 * EP
NA = 10240
NP = 10240
NC = 2
NS = 16
EB = 512
NB = 1000
EBP = 2048
F32 = jnp.float32


def _sc_gather_body(src3, table, out, idx_v, buf, gsem):
    c = lax.axis_index("c")
    s = lax.axis_index("s")
    w = c * NS + s
    pltpu.sync_copy(src3.at[w], idx_v)
    for j in range(25):
        pltpu.async_copy(table.at[idx_v.at[j]], buf, gsem).wait()
        pltpu.sync_copy(buf, out.at[pl.ds((w * 25 + j) * 128, 128)])


@functools.lru_cache(maxsize=None)
def _build_gather():
    return pl.kernel(
        _sc_gather_body,
        out_type=jax.ShapeDtypeStruct((R, D), F32),
        mesh=plsc.VectorSubcoreMesh(core_axis_name="c", subcore_axis_name="s"),
        compiler_params=pltpu.CompilerParams(use_tc_tiling_on_sc=False),
        scratch_types=[
            pltpu.VMEM((25, 128), jnp.int32),
            pltpu.VMEM((128, D), F32),
            pltpu.SemaphoreType.DMA,
        ],
    )


def _sc_gather(src2, table):
    return _build_gather()(src2, table)


def _sc_scatter_body(msgf, dst5, zrows, out, idx_v, rows_v, agg_sp):
    c = lax.axis_index("c")
    s = lax.axis_index("s")
    for g in range(5):
        pltpu.sync_copy(zrows.at[pl.ds(s * 640, 640)],
                        agg_sp.at[pl.ds(s * 640, 640)])
        plsc.subcore_barrier()
        rb = (g * 32 + c * 16 + s) * 640
        pltpu.sync_copy(dst5.at[g, c, s], idx_v)
        pltpu.sync_copy(msgf.at[pl.ds(rb, 640)], rows_v)
        for j in range(5):
            pltpu.sync_copy(rows_v.at[pl.ds(j * 128, 128)],
                            agg_sp.at[idx_v.at[j]], add=True)
        plsc.subcore_barrier()
        pltpu.sync_copy(agg_sp.at[pl.ds(s * 640, 640)],
                        out.at[c, g, pl.ds(s * 640, 640)])


@functools.lru_cache(maxsize=None)
def _build_scatter():
    return pl.kernel(
        _sc_scatter_body,
        out_type=jax.ShapeDtypeStruct((NC, 5, NA, D), F32),
        mesh=plsc.VectorSubcoreMesh(core_axis_name="c", subcore_axis_name="s"),
        compiler_params=pltpu.CompilerParams(use_tc_tiling_on_sc=False),
        scratch_types=[
            pltpu.VMEM((5, 128), jnp.int32),
            pltpu.VMEM((640, D), F32),
            pltpu.VMEM_SHARED((NA, D), F32),
        ],
    )


def _sc_scatter(msgf, dst5, zrows):
    return _build_scatter()(msgf, dst5, zrows)


def _h0_body(x_ref, wp_ref, bp_ref, o_ref):
    o_ref[0] = jax.nn.relu(
        jnp.dot(x_ref[0], wp_ref[0], preferred_element_type=F32) + bp_ref[0])


def _eht_body(ef_ref, w1_ref, b1_ref, o_ref):
    eh = jax.nn.relu(
        jnp.dot(ef_ref[0], w1_ref[0], preferred_element_type=F32) + b1_ref[0])
    o_ref[0] = jnp.transpose(eh, (1, 0))


def _msg_body(ns_ref, eht_ref, w2t_ref, be2_ref, o_ref):
    nst = jnp.transpose(ns_ref[0], (1, 0))
    wet = jnp.dot(w2t_ref[0], eht_ref[0], preferred_element_type=F32)
    msgt = nst[0:1, :] * wet[0:D, :]
    for i in range(1, D):
        msgt = msgt + nst[i:i + 1, :] * wet[i * D:(i + 1) * D, :]
    o_ref[0] = (jnp.transpose(msgt, (1, 0))
                + jnp.dot(ns_ref[0], be2_ref[0], preferred_element_type=F32))


def _gru_body(a0_ref, a1_ref, h_ref, bc_ref, wih_ref, whh_ref, bih_ref,
              bhh_ref, o_ref):
    node = jax.nn.relu(a0_ref[0, 0] + a1_ref[0, 0] + bc_ref[0])
    gi = jnp.dot(node, wih_ref[0], preferred_element_type=F32) + bih_ref[0]
    gh = jnp.dot(h_ref[0], whh_ref[0], preferred_element_type=F32) + bhh_ref[0]
    r = jax.nn.sigmoid(gi[:, :D] + gh[:, :D])
    z = jax.nn.sigmoid(gi[:, D:2 * D] + gh[:, D:2 * D])
    n = jnp.tanh(gi[:, 2 * D:] + r * gh[:, 2 * D:])
    o_ref[0] = (1.0 - z) * n + z * h_ref[0]


def _s2s_body(feat_ref, gr_ref, gc_ref, wih0_ref, wih1_ref, wih2_ref,
              whh0_ref, whh1_ref, whh2_ref, bs_ref, o_ref):
    feat = feat_ref[0]
    lane = lax.broadcasted_iota(jnp.int32, (NP, B), 1).astype(F32)
    sub = lax.broadcasted_iota(jnp.int32, (B, NP), 0).astype(F32)
    onehot = gc_ref[0] == lane
    maskt = (sub == gr_ref[0]).astype(F32)
    wih = [wih0_ref[...], wih1_ref[...], wih2_ref[...]]
    whh = [whh0_ref[...], whh1_ref[...], whh2_ref[...]]
    q_star = jnp.zeros((B, 2 * D), F32)
    h = [jnp.zeros((B, D), F32) for _ in range(3)]
    c = [jnp.zeros((B, D), F32) for _ in range(3)]
    for _ in range(N_S2S):
        inp = q_star
        for l in range(3):
            g = (jnp.dot(inp, wih[l], preferred_element_type=F32)
                 + jnp.dot(h[l], whh[l], preferred_element_type=F32)
                 + bs_ref[l:l + 1, :])
            ii = jax.nn.sigmoid(g[:, :D])
            ff = jax.nn.sigmoid(g[:, D:2 * D])
            gg = jnp.tanh(g[:, 2 * D:3 * D])
            oo = jax.nn.sigmoid(g[:, 3 * D:])
            c[l] = ff * c[l] + ii * gg
            h[l] = oo * jnp.tanh(c[l])
            inp = h[l]
        q = h[2]
        s_all = jnp.dot(feat, jnp.transpose(q, (1, 0)),
                        preferred_element_type=F32)
        emax = jnp.max(jnp.where(onehot, s_all, -1e30), axis=0, keepdims=True)
        ex = jnp.where(onehot, jnp.exp(s_all - emax), 0.0)
        den = jnp.sum(ex, axis=0, keepdims=True)
        den = jnp.where(den > 0.0, den, 1.0)
        alpha = jnp.sum(ex / den, axis=1, keepdims=True)
        readout = jnp.dot(maskt, feat * alpha, preferred_element_type=F32)
        q_star = jnp.concatenate([q, readout], axis=1)
    o_ref[0] = q_star


def _final_body(qs_ref, fa_ref, fb_ref, w1_ref, b1_ref, w2_ref, b2_ref, o_ref):
    ga = fa_ref[:, 0:1] * qs_ref[1] + fa_ref[:, 1:2] * qs_ref[2]
    gb = fb_ref[:, 0:1] * qs_ref[3] + fb_ref[:, 1:2] * qs_ref[4]
    hcat = jnp.concatenate([qs_ref[0], ga, gb], axis=1)
    hid = jax.nn.relu(
        jnp.dot(hcat, w1_ref[...], preferred_element_type=F32) + b1_ref[...])
    o_ref[...] = jnp.dot(hid, w2_ref[...], preferred_element_type=F32) + b2_ref[...]


def _tc(body, grid, in_specs, out_specs, out_shape):
    return pl.pallas_call(body, grid=grid, in_specs=in_specs,
                          out_specs=out_specs, out_shape=out_shape)


def kernel(x_solu, e_solu, x_a1, e_a1, x_a2, e_a2, x_b1, e_b1, x_b2, e_b2,
           facs_a, facs_b, params, ei_solu, gid_solu, ei_a1, gid_a1, ei_a2,
           gid_a2, ei_b1, gid_b1, ei_b2, gid_b2):
    gps = [params['solu'], params['solv_a'], params['solv_a'],
           params['solv_b'], params['solv_b']]
    xs = [x_solu, x_a1, x_a2, x_b1, x_b2]
    efs = [e_solu, e_a1, e_a2, e_b1, e_b2]
    eis = [ei_solu, ei_a1, ei_a2, ei_b1, ei_b2]
    gids = [gid_solu, gid_a1, gid_a2, gid_b1, gid_b2]

    x_all = jnp.stack(xs)
    ef_all = jnp.pad(jnp.stack(efs), ((0, 0), (0, EP - E), (0, 0)))
    wp_all = jnp.stack([gp['Wp'] for gp in gps])
    bp_all = jnp.stack([gp['bp'] for gp in gps])[:, None, :]
    we1_all = jnp.stack([gp['We1'] for gp in gps])
    be1_all = jnp.stack([gp['be1'] for gp in gps])[:, None, :]
    w2t_all = jnp.stack([gp['We2'].T for gp in gps])
    be2_all = jnp.stack([gp['be2'].reshape(D, D) for gp in gps])
    bc_all = jnp.stack([gp['bconv'] for gp in gps])[:, None, :]
    wih_all = jnp.stack([gp['Wih'].T for gp in gps])
    whh_all = jnp.stack([gp['Whh'].T for gp in gps])
    bih_all = jnp.stack([gp['bih'] for gp in gps])[:, None, :]
    bhh_all = jnp.stack([gp['bhh'] for gp in gps])[:, None, :]

    src3 = jnp.stack([
        jnp.pad(eis[g][0], (0, EP - E)) + g * N for g in range(5)
    ]).reshape(NC * NS, 25, 128)
    dst5 = jnp.stack([
        jnp.pad(eis[g][1], (0, EP - E), constant_values=N) for g in range(5)
    ]).reshape(5, NC, NS, 5, 128)
    zrows = jnp.zeros((NA, D), F32)

    sp = params['s2s']
    wih_s = [sp['Wih_%d' % l].T for l in range(3)]
    whh_s = [sp['Whh_%d' % l].T for l in range(3)]
    bs_all = jnp.stack([sp['bih_%d' % l] + sp['bhh_%d' % l] for l in range(3)])

    gidp = jnp.stack([
        jnp.pad(gids[g], (0, NP - N), constant_values=1000) for g in range(5)
    ]).astype(F32)
    gid_row = gidp[:, None, :]
    gid_col = gidp[:, :, None]

    h0 = _tc(_h0_body, (5, N // NB),
             [pl.BlockSpec((1, NB, D_NODE), lambda g, nb: (g, nb, 0)),
              pl.BlockSpec((1, D_NODE, D), lambda g, nb: (g, 0, 0)),
              pl.BlockSpec((1, 1, D), lambda g, nb: (g, 0, 0))],
             pl.BlockSpec((1, NB, D), lambda g, nb: (g, nb, 0)),
             jax.ShapeDtypeStruct((5, N, D), F32))(x_all, wp_all, bp_all)

    eht = _tc(_eht_body, (5, EP // EBP),
              [pl.BlockSpec((1, EBP, 16), lambda g, eb: (g, eb, 0)),
               pl.BlockSpec((1, 16, D_EH), lambda g, eb: (g, 0, 0)),
               pl.BlockSpec((1, 1, D_EH), lambda g, eb: (g, 0, 0))],
              pl.BlockSpec((1, D_EH, EBP), lambda g, eb: (g, 0, eb)),
              jax.ShapeDtypeStruct((5, D_EH, EP), F32))(ef_all, we1_all, be1_all)

    msg_call = _tc(
        _msg_body, (5, EP // EB),
        [pl.BlockSpec((1, EB, D), lambda g, eb: (g, eb, 0)),
         pl.BlockSpec((1, D_EH, EB), lambda g, eb: (g, 0, eb)),
         pl.BlockSpec((1, D * D, D_EH), lambda g, eb: (g, 0, 0)),
         pl.BlockSpec((1, D, D), lambda g, eb: (g, 0, 0))],
        pl.BlockSpec((1, EB, D), lambda g, eb: (g, eb, 0)),
        jax.ShapeDtypeStruct((5, EP, D), F32))

    gru_call = _tc(
        _gru_body, (5, N // NB),
        [pl.BlockSpec((1, 1, NB, D), lambda g, nb: (0, g, nb, 0)),
         pl.BlockSpec((1, 1, NB, D), lambda g, nb: (1, g, nb, 0)),
         pl.BlockSpec((1, NB, D), lambda g, nb: (g, nb, 0)),
         pl.BlockSpec((1, 1, D), lambda g, nb: (g, 0, 0)),
         pl.BlockSpec((1, D, 3 * D), lambda g, nb: (g, 0, 0)),
         pl.BlockSpec((1, D, 3 * D), lambda g, nb: (g, 0, 0)),
         pl.BlockSpec((1, 1, 3 * D), lambda g, nb: (g, 0, 0)),
         pl.BlockSpec((1, 1, 3 * D), lambda g, nb: (g, 0, 0))],
        pl.BlockSpec((1, NB, D), lambda g, nb: (g, nb, 0)),
        jax.ShapeDtypeStruct((5, N, D), F32))

    hidden = h0
    for _ in range(N_MP):
        ns = _sc_gather(src3, hidden.reshape(5 * N, D))
        msg = msg_call(ns.reshape(5, EP, D), eht, w2t_all, be2_all)
        agg2 = _sc_scatter(msg.reshape(R, D), dst5, zrows)
        hidden = gru_call(agg2, agg2, hidden, bc_all, wih_all, whh_all,
                          bih_all, bhh_all)

    feat = jnp.pad(hidden, ((0, 0), (0, NP - N), (0, 0)))
    qstar = _tc(
        _s2s_body, (5,),
        [pl.BlockSpec((1, NP, D), lambda g: (g, 0, 0)),
         pl.BlockSpec((1, 1, NP), lambda g: (g, 0, 0)),
         pl.BlockSpec((1, NP, 1), lambda g: (g, 0, 0)),
         pl.BlockSpec((2 * D, 4 * D), lambda g: (0, 0)),
         pl.BlockSpec((D, 4 * D), lambda g: (0, 0)),
         pl.BlockSpec((D, 4 * D), lambda g: (0, 0)),
         pl.BlockSpec((D, 4 * D), lambda g: (0, 0)),
         pl.BlockSpec((D, 4 * D), lambda g: (0, 0)),
         pl.BlockSpec((D, 4 * D), lambda g: (0, 0)),
         pl.BlockSpec((3, 4 * D), lambda g: (0, 0))],
        pl.BlockSpec((1, B, 2 * D), lambda g: (g, 0, 0)),
        jax.ShapeDtypeStruct((5, B, 2 * D), F32))(
            feat, gid_row, gid_col, wih_s[0], wih_s[1], wih_s[2],
            whh_s[0], whh_s[1], whh_s[2], bs_all)

    pr = params['pred']
    out = pl.pallas_call(
        _final_body,
        out_shape=jax.ShapeDtypeStruct((B, 1), F32))(
            qstar, facs_a, facs_b, pr['W1'], pr['b1'][None, :], pr['W2'],
            pr['b2'][None, :])
    return out
